# baseline (speedup 1.0000x reference)
import jax
import jax.numpy as jnp
from jax.experimental import pallas as pl
from jax.experimental.pallas import tpu as pltpu

B, C, HW = 1024, 3, 64
D = C * HW * HW
Z, H, NL = 128, 1024, 17
KT = 512
NKA = D // KT
MT = 64
NM = B // MT
NT = 1024
NNB = D // NT


def _fused(xf_ref, we_ref, y_ref, eps_ref, wtail_ref, be_ref, wmu_ref,
           bmu_ref, wls_ref, bls_ref, wdz_ref, wdy_ref, bd_ref, wo_ref,
           bo_ref, out_ref, wbf_ref, h2s_ref):
    k = pl.program_id(0)

    @pl.when(k < NKA)
    def _load_w():
        wbf_ref[pl.ds(k * KT, KT), :] = we_ref[...].astype(jnp.bfloat16)

    @pl.when(jnp.logical_and(k >= NKA, k < NKA + NM))
    def _encode():
        xb = xf_ref[...].astype(jnp.bfloat16)
        yb = y_ref[...].astype(jnp.bfloat16)
        pre = (jnp.dot(xb, wbf_ref[...], preferred_element_type=jnp.float32)
               + be_ref[...]
               + jnp.dot(yb, wtail_ref[...].astype(jnp.bfloat16),
                         preferred_element_type=jnp.float32))
        h = jax.nn.softplus(pre)
        hb = h.astype(jnp.bfloat16)
        z_loc = (jnp.dot(hb, wmu_ref[...].astype(jnp.bfloat16),
                         preferred_element_type=jnp.float32) + bmu_ref[...])
        z_ls = (jnp.dot(hb, wls_ref[...].astype(jnp.bfloat16),
                        preferred_element_type=jnp.float32) + bls_ref[...])
        z = z_loc + jnp.exp(z_ls) * eps_ref[...]
        pre2 = (jnp.dot(z.astype(jnp.bfloat16), wdz_ref[...].astype(jnp.bfloat16),
                        preferred_element_type=jnp.float32)
                + jnp.dot(yb, wdy_ref[...].astype(jnp.bfloat16),
                          preferred_element_type=jnp.float32)
                + bd_ref[...])
        h2s_ref[pl.ds((k - NKA) * MT, MT), :] = (
            jax.nn.softplus(pre2).astype(jnp.bfloat16))

    @pl.when(k >= NKA + NM)
    def _decode():
        acc = jnp.dot(h2s_ref[...], wo_ref[...].astype(jnp.bfloat16),
                      preferred_element_type=jnp.float32)
        out_ref[...] = 0.5 * jnp.tanh(0.5 * (acc + bo_ref[...])) + 0.5


def kernel(x, y, eps, W_e1, b_e1, W_mu, b_mu, W_ls, b_ls, W_d1, b_d1, W_out, b_out):
    n = x.shape[0]
    xf = x.reshape(n, D)
    W_tail = jax.lax.slice(W_e1, (D, 0), (D + NL, H))
    W_dz = jax.lax.slice(W_d1, (0, 0), (Z, H))
    W_dy = jax.lax.slice(W_d1, (Z, 0), (Z + NL, H))

    full = lambda shape: pl.BlockSpec(shape, lambda k: (0,) * len(shape))
    mrow = lambda w: pl.BlockSpec(
        (MT, w), lambda k: (jnp.clip(k - NKA, 0, NM - 1), 0))
    ntile = lambda h_: pl.BlockSpec(
        (h_, NT), lambda k: (0, jnp.clip(k - NKA - NM, 0, NNB - 1)))

    out = pl.pallas_call(
        _fused,
        grid=(NKA + NM + NNB,),
        in_specs=[
            mrow(D),
            pl.BlockSpec((KT, H), lambda k: (jnp.minimum(k, NKA - 1), 0)),
            mrow(NL),
            mrow(Z),
            full((NL, H)),
            full((1, H)),
            full((H, Z)),
            full((1, Z)),
            full((H, Z)),
            full((1, Z)),
            full((Z, H)),
            full((NL, H)),
            full((1, H)),
            ntile(H),
            ntile(1),
        ],
        out_specs=pl.BlockSpec(
            (n, NT), lambda k: (0, jnp.clip(k - NKA - NM, 0, NNB - 1))),
        out_shape=jax.ShapeDtypeStruct((n, D), jnp.float32),
        scratch_shapes=[pltpu.VMEM((D, H), jnp.bfloat16),
                        pltpu.VMEM((B, H), jnp.bfloat16)],
        compiler_params=pltpu.CompilerParams(
            dimension_semantics=("arbitrary",),
            vmem_limit_bytes=110 * 1024 * 1024,
        ),
    )(xf, W_e1, y, eps, W_tail, b_e1.reshape(1, H), W_mu, b_mu.reshape(1, Z),
      W_ls, b_ls.reshape(1, Z), W_dz, W_dy, b_d1.reshape(1, H),
      W_out, b_out.reshape(1, D))

    return out.reshape(x.shape)


# speedup vs baseline: 1.3099x; 1.3099x over previous
import jax
import jax.numpy as jnp
from jax.experimental import pallas as pl
from jax.experimental.pallas import tpu as pltpu

B, C, HW = 1024, 3, 64
D = C * HW * HW
Z, H, NL = 128, 1024, 17
KT = 1024
NK = D // KT
NT = 1024
NN = D // NT


def _stage_a(xf_ref, we_ref, y_ref, eps_ref, wtail_ref, be_ref, wmu_ref,
             bmu_ref, wls_ref, bls_ref, wdz_ref, wdy_ref, bd_ref,
             h2_ref, acc_ref):
    k = pl.program_id(0)

    @pl.when(k == 0)
    def _init():
        acc_ref[...] = jnp.zeros_like(acc_ref)

    acc_ref[...] += jnp.dot(xf_ref[...].astype(jnp.bfloat16),
                            we_ref[...].astype(jnp.bfloat16),
                            preferred_element_type=jnp.float32)

    @pl.when(k == NK - 1)
    def _tail():
        yb = y_ref[...].astype(jnp.bfloat16)
        pre = (acc_ref[...] + be_ref[...]
               + jnp.dot(yb, wtail_ref[...].astype(jnp.bfloat16),
                         preferred_element_type=jnp.float32))
        h = jax.nn.softplus(pre)
        hb = h.astype(jnp.bfloat16)
        z_loc = (jnp.dot(hb, wmu_ref[...].astype(jnp.bfloat16),
                         preferred_element_type=jnp.float32) + bmu_ref[...])
        z_ls = (jnp.dot(hb, wls_ref[...].astype(jnp.bfloat16),
                        preferred_element_type=jnp.float32) + bls_ref[...])
        z = z_loc + jnp.exp(z_ls) * eps_ref[...]
        pre2 = (jnp.dot(z.astype(jnp.bfloat16),
                        wdz_ref[...].astype(jnp.bfloat16),
                        preferred_element_type=jnp.float32)
                + jnp.dot(yb, wdy_ref[...].astype(jnp.bfloat16),
                          preferred_element_type=jnp.float32)
                + bd_ref[...])
        h2_ref[...] = jax.nn.softplus(pre2).astype(jnp.bfloat16)


def _stage_b(h2_ref, wo_ref, bo_ref, out_ref):
    acc = jnp.dot(h2_ref[...], wo_ref[...].astype(jnp.bfloat16),
                  preferred_element_type=jnp.float32)
    out_ref[...] = 0.5 * jnp.tanh(0.5 * (acc + bo_ref[...])) + 0.5


def kernel(x, y, eps, W_e1, b_e1, W_mu, b_mu, W_ls, b_ls, W_d1, b_d1, W_out, b_out):
    n = x.shape[0]
    xf = x.reshape(n, D)
    W_tail = jax.lax.slice(W_e1, (D, 0), (D + NL, H))
    W_dz = jax.lax.slice(W_d1, (0, 0), (Z, H))
    W_dy = jax.lax.slice(W_d1, (Z, 0), (Z + NL, H))

    full = lambda shape: pl.BlockSpec(shape, lambda k: (0,) * len(shape))

    h2 = pl.pallas_call(
        _stage_a,
        grid=(NK,),
        in_specs=[
            pl.BlockSpec((n, KT), lambda k: (0, k)),
            pl.BlockSpec((KT, H), lambda k: (k, 0)),
            full((n, NL)),
            full((n, Z)),
            full((NL, H)),
            full((1, H)),
            full((H, Z)),
            full((1, Z)),
            full((H, Z)),
            full((1, Z)),
            full((Z, H)),
            full((NL, H)),
            full((1, H)),
        ],
        out_specs=full((n, H)),
        out_shape=jax.ShapeDtypeStruct((n, H), jnp.bfloat16),
        scratch_shapes=[pltpu.VMEM((n, H), jnp.float32)],
        compiler_params=pltpu.CompilerParams(
            dimension_semantics=("arbitrary",),
            vmem_limit_bytes=110 * 1024 * 1024,
        ),
    )(xf, W_e1, y, eps, W_tail, b_e1.reshape(1, H), W_mu, b_mu.reshape(1, Z),
      W_ls, b_ls.reshape(1, Z), W_dz, W_dy, b_d1.reshape(1, H))

    out = pl.pallas_call(
        _stage_b,
        grid=(NN,),
        in_specs=[
            full((n, H)),
            pl.BlockSpec((H, NT), lambda j: (0, j)),
            pl.BlockSpec((1, NT), lambda j: (0, j)),
        ],
        out_specs=pl.BlockSpec((n, NT), lambda j: (0, j)),
        out_shape=jax.ShapeDtypeStruct((n, D), jnp.float32),
        compiler_params=pltpu.CompilerParams(
            dimension_semantics=("arbitrary",),
            vmem_limit_bytes=110 * 1024 * 1024,
        ),
    )(h2, W_out, b_out.reshape(1, D))

    return out.reshape(x.shape)
